# Initial kernel scaffold; baseline (speedup 1.0000x reference)
import jax
import jax.numpy as jnp
from jax.experimental import pallas as pl


def kernel(x, edge_index, W_conv1, b_conv1, W_lin, b_lin, W_gate1, b_gate1, bn1_w, bn1_b, W_conv2, b_conv2, W_gate2, b_gate2, bn2_w, bn2_b):
    raise NotImplementedError("write your pallas kernel here")


# speedup vs baseline: 2.5711x; 2.5711x over previous
import jax
import jax.numpy as jnp
from jax import lax
from jax.experimental import pallas as pl
from jax.experimental.pallas import tpu as pltpu
from jax.experimental.pallas import tpu_sc as plsc

_N = 10000
_E = 160000
_DIN = 256
_DH = 512
_DHH = _DH // 2

_NC, _NS = 2, 16
_NT = _NC * _NS
_NPAD = 10240
_EPAD = 163840
_SUB = 2560
_NSUB = _EPAD // _SUB
_RPT = _NPAD // _NT
_BATCH = 32
_LCAP = _EPAD + _NSUB * 8 + 64
_RB = 256
_GRID = _NPAD // _RB

_f32 = jnp.float32
_i32 = jnp.int32


def _mesh():
    return plsc.VectorSubcoreMesh(core_axis_name="c", subcore_axis_name="s")


def _index_body(src_hbm, dst_hbm, lsrc_hbm, lldst_hbm, cnt_hbm, parts_hbm,
                esrc_v, edst_v, ssrc_v, sdst_v, hist_v, cnt_v):
    c = lax.axis_index("c")
    s = lax.axis_index("s")
    t = c * _NS + s
    lo = t * _RPT

    def zh(i, carry):
        hist_v[pl.ds(i * 16, 16)] = jnp.zeros((16,), _f32)
        return carry

    lax.fori_loop(0, _NPAD // 16, zh, 0)

    onesf = jnp.ones((16,), _f32)
    pad_d = jnp.full((16,), _RPT, _i32)
    pad_s = jnp.zeros((16,), _i32)
    degr0 = t * (_NSUB // _NT)

    def round_body(r, off):
        off = pl.multiple_of(off, 8)
        rb = pl.multiple_of(r * _SUB, 8)
        pltpu.sync_copy(src_hbm.at[pl.ds(rb, _SUB)], esrc_v)
        pltpu.sync_copy(dst_hbm.at[pl.ds(rb, _SUB)], edst_v)

        @pl.when((r >= degr0) & (r < degr0 + _NSUB // _NT))
        def _():
            def dbody(i, carry):
                d = edst_v[pl.ds(i * 16, 16)]
                plsc.addupdate_scatter(hist_v, [d], onesf)
                return carry

            lax.fori_loop(0, _SUB // 16, dbody, 0)

        def fbody(i, loff):
            d = edst_v[pl.ds(i * 16, 16)]
            sv = esrc_v[pl.ds(i * 16, 16)]
            m = (d >= lo) & (d < lo + _RPT)
            plsc.store_compressed(ssrc_v.at[pl.ds(loff, 16)], sv, mask=m)
            plsc.store_compressed(sdst_v.at[pl.ds(loff, 16)], d - lo, mask=m)
            return loff + jnp.sum(jnp.where(m, 1, 0))

        loff = lax.fori_loop(0, _SUB // 16, fbody, 0)
        sdst_v[pl.ds(loff, 16)] = pad_d
        ssrc_v[pl.ds(loff, 16)] = pad_s
        loff8 = (loff + 7) // 8 * 8
        ob = pl.multiple_of(t * _LCAP + off, 8)
        pltpu.sync_copy(ssrc_v, lsrc_hbm.at[pl.ds(ob, _SUB + 16)])
        pltpu.sync_copy(sdst_v, lldst_hbm.at[pl.ds(ob, _SUB + 16)])
        return off + loff8

    cnt = lax.fori_loop(0, _NSUB, round_body, 0)

    for j in range(_BATCH // 16):
        sdst_v[pl.ds(j * 16, 16)] = pad_d
        ssrc_v[pl.ds(j * 16, 16)] = pad_s
    cb = pl.multiple_of(t * _LCAP + cnt, 8)
    pltpu.sync_copy(ssrc_v.at[pl.ds(0, _BATCH)],
                    lsrc_hbm.at[pl.ds(cb, _BATCH)])
    pltpu.sync_copy(sdst_v.at[pl.ds(0, _BATCH)],
                    lldst_hbm.at[pl.ds(cb, _BATCH)])

    cnt_v[pl.ds(0, 16)] = jnp.full((16,), cnt, _i32)
    pltpu.sync_copy(cnt_v.at[pl.ds(0, 8)],
                    cnt_hbm.at[pl.ds(pl.multiple_of(t * 8, 8), 8)])
    pltpu.sync_copy(hist_v, parts_hbm.at[t])


def _index_call(src_p, dst_p):
    return pl.kernel(
        _index_body,
        out_type=(
            jax.ShapeDtypeStruct((_NT * _LCAP,), _i32),
            jax.ShapeDtypeStruct((_NT * _LCAP,), _i32),
            jax.ShapeDtypeStruct((_NT * 8,), _i32),
            jax.ShapeDtypeStruct((_NT, _NPAD), _f32),
        ),
        mesh=_mesh(),
        compiler_params=pltpu.CompilerParams(needs_layout_passes=False),
        scratch_types=[
            pltpu.VMEM((_SUB,), _i32),
            pltpu.VMEM((_SUB,), _i32),
            pltpu.VMEM((_SUB + 16,), _i32),
            pltpu.VMEM((_SUB + 16,), _i32),
            pltpu.VMEM((_NPAD,), _f32),
            pltpu.VMEM((16,), _i32),
        ],
    )(src_p, dst_p)


def _scatter_body(h2_hbm, lsrc_hbm, lldst_hbm, cnt_hbm, outa_hbm, outb_hbm,
                  lsrc_v, lldst_v, idx_v, rows_v, cnt_v, acc_v, sem):
    c = lax.axis_index("c")
    s = lax.axis_index("s")
    t = c * _NS + s
    lo = t * _RPT

    pltpu.sync_copy(cnt_hbm.at[pl.ds(pl.multiple_of(t * 8, 8), 8)],
                    cnt_v.at[pl.ds(0, 8)])
    cnt = cnt_v[pl.ds(0, 16)][0]
    nblk = (cnt + _SUB - 1) // _SUB

    for half in range(2):
        out_h = outa_hbm if half == 0 else outb_hbm

        def za(i, carry):
            acc_v[i // (_DHH // 16), pl.ds((i % (_DHH // 16)) * 16, 16)] = (
                jnp.zeros((16,), _f32))
            return carry

        lax.fori_loop(0, (_RPT + 1) * (_DHH // 16), za, 0)

        def blk_body(blk, carry):
            lb = pl.multiple_of(t * _LCAP + blk * _SUB, 8)
            pltpu.sync_copy(lsrc_hbm.at[pl.ds(lb, _SUB)], lsrc_v)
            pltpu.sync_copy(lldst_hbm.at[pl.ds(lb, _SUB)],
                            lldst_v.at[pl.ds(0, _SUB)])
            size = jnp.minimum(_SUB, cnt - blk * _SUB)
            nb = (size + _BATCH - 1) // _BATCH

            def gbody(b, carry2):
                for j in range(_BATCH // 16):
                    sv = lsrc_v[pl.ds(b * _BATCH + j * 16, 16)]
                    idx_v[pl.ds(j * 16, 16)] = sv * 2 + half
                pltpu.async_copy(h2_hbm.at[idx_v], rows_v, sem).wait()

                def ebody(e, carry3):
                    ldst = lldst_v[pl.ds(b * _BATCH + e, 16)][0]
                    for j in range(_DHH // 16):
                        plsc.addupdate(acc_v.at[ldst, pl.ds(j * 16, 16)],
                                       rows_v[e, pl.ds(j * 16, 16)])
                    return carry3

                lax.fori_loop(0, _BATCH, ebody, 0)
                return carry2

            lax.fori_loop(0, nb, gbody, 0)
            return carry

        lax.fori_loop(0, nblk, blk_body, 0)
        pltpu.sync_copy(acc_v.at[pl.ds(0, _RPT)],
                        out_h.at[pl.ds(pl.multiple_of(lo, 8), _RPT)])


def _scatter_call(h_p, lsrc, lldst, cnts):
    h2 = h_p.reshape(2 * _NPAD, _DHH)
    return pl.kernel(
        _scatter_body,
        out_type=(
            jax.ShapeDtypeStruct((_NPAD, _DHH), _f32),
            jax.ShapeDtypeStruct((_NPAD, _DHH), _f32),
        ),
        mesh=_mesh(),
        compiler_params=pltpu.CompilerParams(needs_layout_passes=False),
        scratch_types=[
            pltpu.VMEM((_SUB,), _i32),
            pltpu.VMEM((_SUB + 16,), _i32),
            pltpu.VMEM((_BATCH,), _i32),
            pltpu.VMEM((_BATCH, _DHH), _f32),
            pltpu.VMEM((16,), _i32),
            pltpu.VMEM((_RPT + 1, _DHH), _f32),
            pltpu.SemaphoreType.DMA,
        ],
    )(h2, lsrc, lldst, cnts)


def _dinv_from_parts(degp):
    deg = jnp.sum(degp, axis=0) + 1.0
    return lax.rsqrt(deg)


def _prep_kernel(x_ref, w1_ref, wl_ref, bl_ref, degp_ref, h1p_ref, hlin_ref):
    dinv = _dinv_from_parts(degp_ref[...])[:, None]
    xb = x_ref[...]
    h1 = jnp.dot(xb, w1_ref[...], preferred_element_type=_f32,
                 precision=lax.Precision.HIGHEST)
    h1p_ref[...] = h1 * dinv
    hlin_ref[...] = jnp.dot(xb, wl_ref[...], preferred_element_type=_f32,
                            precision=lax.Precision.HIGHEST) + bl_ref[...]


def _prep_call(x_p, W1, Wl, bl, degp):
    return pl.pallas_call(
        _prep_kernel,
        grid=(_GRID,),
        in_specs=[
            pl.BlockSpec((_RB, _DIN), lambda i: (i, 0)),
            pl.BlockSpec((_DIN, _DH), lambda i: (0, 0)),
            pl.BlockSpec((_DIN, _DH), lambda i: (0, 0)),
            pl.BlockSpec((1, _DH), lambda i: (0, 0)),
            pl.BlockSpec((_NT, _RB), lambda i: (0, i)),
        ],
        out_specs=[
            pl.BlockSpec((_RB, _DH), lambda i: (i, 0)),
            pl.BlockSpec((_RB, _DH), lambda i: (i, 0)),
        ],
        out_shape=[
            jax.ShapeDtypeStruct((_NPAD, _DH), _f32),
            jax.ShapeDtypeStruct((_NPAD, _DH), _f32),
        ],
    )(x_p, W1, Wl, bl, degp)


def _gate_kernel(Sa_ref, Sb_ref, hp_ref, res_ref, degp_ref, bc_ref, wg_ref,
                 bg_ref, hpre_ref, sum_ref, sq_ref, acc_s, acc_q):
    i = pl.program_id(0)
    dinv = _dinv_from_parts(degp_ref[...])[:, None]
    S = jnp.concatenate([Sa_ref[...], Sb_ref[...]], axis=1)
    z = jnp.tanh((S + hp_ref[...]) * dinv + bc_ref[...])
    g = jax.nn.sigmoid(jnp.dot(z, wg_ref[...], preferred_element_type=_f32,
                               precision=lax.Precision.HIGHEST) + bg_ref[...])
    h = jax.nn.relu((1.0 - g) * res_ref[...] + g * z)
    ridx = i * _RB + lax.broadcasted_iota(_i32, (_RB, 1), 0)
    h = jnp.where(ridx < _N, h, 0.0)
    hpre_ref[...] = h

    @pl.when(i == 0)
    def _():
        acc_s[...] = jnp.zeros_like(acc_s)
        acc_q[...] = jnp.zeros_like(acc_q)

    acc_s[...] += jnp.sum(h, axis=0, keepdims=True)
    acc_q[...] += jnp.sum(h * h, axis=0, keepdims=True)
    sum_ref[...] = acc_s[...]
    sq_ref[...] = acc_q[...]


def _gate_call(Sa, Sb, hp, res, degp, bc, Wg, bg):
    return pl.pallas_call(
        _gate_kernel,
        grid=(_GRID,),
        in_specs=[
            pl.BlockSpec((_RB, _DHH), lambda i: (i, 0)),
            pl.BlockSpec((_RB, _DHH), lambda i: (i, 0)),
            pl.BlockSpec((_RB, _DH), lambda i: (i, 0)),
            pl.BlockSpec((_RB, _DH), lambda i: (i, 0)),
            pl.BlockSpec((_NT, _RB), lambda i: (0, i)),
            pl.BlockSpec((1, _DH), lambda i: (0, 0)),
            pl.BlockSpec((_DH, _DH), lambda i: (0, 0)),
            pl.BlockSpec((1, _DH), lambda i: (0, 0)),
        ],
        out_specs=[
            pl.BlockSpec((_RB, _DH), lambda i: (i, 0)),
            pl.BlockSpec((1, _DH), lambda i: (0, 0)),
            pl.BlockSpec((1, _DH), lambda i: (0, 0)),
        ],
        out_shape=[
            jax.ShapeDtypeStruct((_NPAD, _DH), _f32),
            jax.ShapeDtypeStruct((1, _DH), _f32),
            jax.ShapeDtypeStruct((1, _DH), _f32),
        ],
        scratch_shapes=[
            pltpu.VMEM((1, _DH), _f32),
            pltpu.VMEM((1, _DH), _f32),
        ],
    )(Sa, Sb, hp, res, degp, bc, Wg, bg)


def _bnconv_kernel(hpre_ref, sc_ref, sh_ref, degp_ref, w2_ref,
                   hbn_ref, h2p_ref):
    dinv = _dinv_from_parts(degp_ref[...])[:, None]
    hbn = hpre_ref[...] * sc_ref[...] + sh_ref[...]
    i = pl.program_id(0)
    ridx = i * _RB + lax.broadcasted_iota(_i32, (_RB, 1), 0)
    hbn = jnp.where(ridx < _N, hbn, 0.0)
    hbn_ref[...] = hbn
    h2p_ref[...] = dinv * jnp.dot(hbn, w2_ref[...], preferred_element_type=_f32,
                                  precision=lax.Precision.HIGHEST)


def _bnconv_call(hpre, scale, shift, degp, W2):
    return pl.pallas_call(
        _bnconv_kernel,
        grid=(_GRID,),
        in_specs=[
            pl.BlockSpec((_RB, _DH), lambda i: (i, 0)),
            pl.BlockSpec((1, _DH), lambda i: (0, 0)),
            pl.BlockSpec((1, _DH), lambda i: (0, 0)),
            pl.BlockSpec((_NT, _RB), lambda i: (0, i)),
            pl.BlockSpec((_DH, _DH), lambda i: (0, 0)),
        ],
        out_specs=[
            pl.BlockSpec((_RB, _DH), lambda i: (i, 0)),
            pl.BlockSpec((_RB, _DH), lambda i: (i, 0)),
        ],
        out_shape=[
            jax.ShapeDtypeStruct((_NPAD, _DH), _f32),
            jax.ShapeDtypeStruct((_NPAD, _DH), _f32),
        ],
    )(hpre, scale, shift, degp, W2)


def _bnapply_kernel(hpre_ref, sc_ref, sh_ref, out_ref):
    out_ref[...] = hpre_ref[...] * sc_ref[...] + sh_ref[...]


def _bnapply_call(hpre, scale, shift):
    return pl.pallas_call(
        _bnapply_kernel,
        grid=(_GRID,),
        in_specs=[
            pl.BlockSpec((_RB, _DH), lambda i: (i, 0)),
            pl.BlockSpec((1, _DH), lambda i: (0, 0)),
            pl.BlockSpec((1, _DH), lambda i: (0, 0)),
        ],
        out_specs=pl.BlockSpec((_RB, _DH), lambda i: (i, 0)),
        out_shape=jax.ShapeDtypeStruct((_NPAD, _DH), _f32),
    )(hpre, scale, shift)


def kernel(x, edge_index, W_conv1, b_conv1, W_lin, b_lin, W_gate1, b_gate1,
           bn1_w, bn1_b, W_conv2, b_conv2, W_gate2, b_gate2, bn2_w, bn2_b):
    src = edge_index[0].astype(_i32)
    dst = edge_index[1].astype(_i32)
    pad = jnp.full((_EPAD - _E,), _N, _i32)
    src_p = jnp.concatenate([src, pad])
    dst_p = jnp.concatenate([dst, pad])
    x_p = jnp.pad(x, ((0, _NPAD - _N), (0, 0)))

    lsrc, lldst, cnts, degp = _index_call(src_p, dst_p)
    h1p, hlin = _prep_call(x_p, W_conv1, W_lin, b_lin.reshape(1, -1), degp)
    S1a, S1b = _scatter_call(h1p, lsrc, lldst, cnts)
    hpre, s1, q1 = _gate_call(S1a, S1b, h1p, hlin, degp,
                              b_conv1.reshape(1, -1), W_gate1,
                              b_gate1.reshape(1, -1))
    mu = s1 / _N
    var = q1 / _N - mu * mu
    scale1 = bn1_w.reshape(1, -1) * lax.rsqrt(var + 1e-5)
    shift1 = bn1_b.reshape(1, -1) - mu * scale1
    hbn, h2p = _bnconv_call(hpre, scale1, shift1, degp, W_conv2)
    S2a, S2b = _scatter_call(h2p, lsrc, lldst, cnts)
    hpre2, s2, q2 = _gate_call(S2a, S2b, h2p, hbn, degp,
                               b_conv2.reshape(1, -1), W_gate2,
                               b_gate2.reshape(1, -1))
    mu2 = s2 / _N
    var2 = q2 / _N - mu2 * mu2
    scale2 = bn2_w.reshape(1, -1) * lax.rsqrt(var2 + 1e-5)
    shift2 = bn2_b.reshape(1, -1) - mu2 * scale2
    out = _bnapply_call(hpre2, scale2, shift2)
    return out[:_N]


# speedup vs baseline: 3.4722x; 1.3505x over previous
import jax
import jax.numpy as jnp
from jax import lax
from jax.experimental import pallas as pl
from jax.experimental.pallas import tpu as pltpu
from jax.experimental.pallas import tpu_sc as plsc

_N = 10000
_E = 160000
_DIN = 256
_DH = 512
_DHH = _DH // 2

_NC, _NS = 2, 16
_NT = _NC * _NS
_NPAD = 10240
_EPAD = 163840
_SUB = 2560
_NSUB = _EPAD // _SUB
_RPT = _NPAD // _NT
_BATCH = 32
_LCAP = _EPAD + _NSUB * 8 + 64
_RB = 256
_GRID = _NPAD // _RB

_f32 = jnp.float32
_i32 = jnp.int32


def _mesh():
    return plsc.VectorSubcoreMesh(core_axis_name="c", subcore_axis_name="s")


def _index_body(src_hbm, dst_hbm, lsrc_hbm, lldst_hbm, cnt_hbm, parts_hbm,
                esrc_v, edst_v, ssrc_v, sdst_v, hist_v, cnt_v):
    c = lax.axis_index("c")
    s = lax.axis_index("s")
    t = c * _NS + s
    lo = t * _RPT

    def zh(i, carry):
        hist_v[pl.ds(i * 16, 16)] = jnp.zeros((16,), _f32)
        return carry

    lax.fori_loop(0, _NPAD // 16, zh, 0)

    onesf = jnp.ones((16,), _f32)
    pad_d = jnp.full((16,), _RPT, _i32)
    pad_s = jnp.zeros((16,), _i32)
    degr0 = t * (_NSUB // _NT)

    def round_body(r, off):
        off = pl.multiple_of(off, 8)
        rb = pl.multiple_of(r * _SUB, 8)
        pltpu.sync_copy(src_hbm.at[pl.ds(rb, _SUB)], esrc_v)
        pltpu.sync_copy(dst_hbm.at[pl.ds(rb, _SUB)], edst_v)

        @pl.when((r >= degr0) & (r < degr0 + _NSUB // _NT))
        def _():
            def dbody(i, carry):
                d = edst_v[pl.ds(i * 16, 16)]
                plsc.addupdate_scatter(hist_v, [d], onesf)
                return carry

            lax.fori_loop(0, _SUB // 16, dbody, 0)

        def fbody(i, loff):
            d = edst_v[pl.ds(i * 16, 16)]
            sv = esrc_v[pl.ds(i * 16, 16)]
            m = (d >= lo) & (d < lo + _RPT)
            plsc.store_compressed(ssrc_v.at[pl.ds(loff, 16)], sv, mask=m)
            plsc.store_compressed(sdst_v.at[pl.ds(loff, 16)], d - lo, mask=m)
            return loff + jnp.sum(jnp.where(m, 1, 0))

        loff = lax.fori_loop(0, _SUB // 16, fbody, 0)
        sdst_v[pl.ds(loff, 16)] = pad_d
        ssrc_v[pl.ds(loff, 16)] = pad_s
        loff8 = (loff + 7) // 8 * 8
        ob = pl.multiple_of(t * _LCAP + off, 8)
        pltpu.sync_copy(ssrc_v, lsrc_hbm.at[pl.ds(ob, _SUB + 16)])
        pltpu.sync_copy(sdst_v, lldst_hbm.at[pl.ds(ob, _SUB + 16)])
        return off + loff8

    cnt = lax.fori_loop(0, _NSUB, round_body, 0)

    for j in range(_BATCH // 16):
        sdst_v[pl.ds(j * 16, 16)] = pad_d
        ssrc_v[pl.ds(j * 16, 16)] = pad_s
    cb = pl.multiple_of(t * _LCAP + cnt, 8)
    pltpu.sync_copy(ssrc_v.at[pl.ds(0, _BATCH)],
                    lsrc_hbm.at[pl.ds(cb, _BATCH)])
    pltpu.sync_copy(sdst_v.at[pl.ds(0, _BATCH)],
                    lldst_hbm.at[pl.ds(cb, _BATCH)])

    cnt_v[pl.ds(0, 16)] = jnp.full((16,), cnt, _i32)
    pltpu.sync_copy(cnt_v.at[pl.ds(0, 8)],
                    cnt_hbm.at[pl.ds(pl.multiple_of(t * 8, 8), 8)])
    pltpu.sync_copy(hist_v, parts_hbm.at[t])


def _index_call(src_p, dst_p):
    return pl.kernel(
        _index_body,
        out_type=(
            jax.ShapeDtypeStruct((_NT * _LCAP,), _i32),
            jax.ShapeDtypeStruct((_NT * _LCAP,), _i32),
            jax.ShapeDtypeStruct((_NT * 8,), _i32),
            jax.ShapeDtypeStruct((_NT, _NPAD), _f32),
        ),
        mesh=_mesh(),
        compiler_params=pltpu.CompilerParams(needs_layout_passes=False),
        scratch_types=[
            pltpu.VMEM((_SUB,), _i32),
            pltpu.VMEM((_SUB,), _i32),
            pltpu.VMEM((_SUB + 16,), _i32),
            pltpu.VMEM((_SUB + 16,), _i32),
            pltpu.VMEM((_NPAD,), _f32),
            pltpu.VMEM((16,), _i32),
        ],
    )(src_p, dst_p)


def _scatter_body(h2_hbm, lsrc_hbm, lldst_hbm, cnt_hbm, outa_hbm, outb_hbm,
                  lsrc_v, lldst_v, idx0_v, idx1_v, rows0_v, rows1_v, cnt_v,
                  acc_v, sem0, sem1):
    c = lax.axis_index("c")
    s = lax.axis_index("s")
    t = c * _NS + s
    lo = t * _RPT
    idx_bufs = (idx0_v, idx1_v)
    rows_bufs = (rows0_v, rows1_v)
    sems = (sem0, sem1)

    pltpu.sync_copy(cnt_hbm.at[pl.ds(pl.multiple_of(t * 8, 8), 8)],
                    cnt_v.at[pl.ds(0, 8)])
    cnt = cnt_v[pl.ds(0, 16)][0]
    nblk = (cnt + _SUB - 1) // _SUB

    for half in range(2):
        out_h = outa_hbm if half == 0 else outb_hbm

        def za(i, carry):
            acc_v[i // (_DHH // 16), pl.ds((i % (_DHH // 16)) * 16, 16)] = (
                jnp.zeros((16,), _f32))
            return carry

        lax.fori_loop(0, (_RPT + 1) * (_DHH // 16), za, 0)

        def blk_body(blk, carry):
            lb = pl.multiple_of(t * _LCAP + blk * _SUB, 8)
            pltpu.sync_copy(lsrc_hbm.at[pl.ds(lb, _SUB)], lsrc_v)
            pltpu.sync_copy(lldst_hbm.at[pl.ds(lb, _SUB)],
                            lldst_v.at[pl.ds(0, _SUB)])
            size = jnp.minimum(_SUB, cnt - blk * _SUB)
            nb = (size + _BATCH - 1) // _BATCH

            def stage_start(b, par):
                ib, sm = idx_bufs[par], sems[par]
                for j in range(_BATCH // 16):
                    sv = lsrc_v[pl.ds(b * _BATCH + j * 16, 16)]
                    ib[pl.ds(j * 16, 16)] = sv * 2 + half
                pltpu.async_copy(h2_hbm.at[ib], rows_bufs[par], sm)

            def wait_process(b, par):
                rb2 = rows_bufs[par]
                pltpu.make_async_copy(h2_hbm.at[idx_bufs[par]], rb2,
                                      sems[par]).wait()
                for e in range(_BATCH):
                    ldst = lldst_v[pl.ds(b * _BATCH + e, 16)][0]
                    for j in range(_DHH // 16):
                        plsc.addupdate(acc_v.at[ldst, pl.ds(j * 16, 16)],
                                       rb2[e, pl.ds(j * 16, 16)])

            stage_start(0, 0)
            ng = (nb + 1) // 2

            def gb(g, carry2):
                for par in range(2):
                    b = g * 2 + par

                    @pl.when(b < nb)
                    def _():
                        @pl.when(b + 1 < nb)
                        def _():
                            stage_start(b + 1, 1 - par)

                        wait_process(b, par)

                return carry2

            lax.fori_loop(0, ng, gb, 0)
            return carry

        lax.fori_loop(0, nblk, blk_body, 0)
        pltpu.sync_copy(acc_v.at[pl.ds(0, _RPT)],
                        out_h.at[pl.ds(pl.multiple_of(lo, 8), _RPT)])


def _scatter_call(h_p, lsrc, lldst, cnts):
    h2 = h_p.reshape(2 * _NPAD, _DHH)
    return pl.kernel(
        _scatter_body,
        out_type=(
            jax.ShapeDtypeStruct((_NPAD, _DHH), _f32),
            jax.ShapeDtypeStruct((_NPAD, _DHH), _f32),
        ),
        mesh=_mesh(),
        compiler_params=pltpu.CompilerParams(needs_layout_passes=False),
        scratch_types=[
            pltpu.VMEM((_SUB,), _i32),
            pltpu.VMEM((_SUB + 16,), _i32),
            pltpu.VMEM((_BATCH,), _i32),
            pltpu.VMEM((_BATCH,), _i32),
            pltpu.VMEM((_BATCH, _DHH), _f32),
            pltpu.VMEM((_BATCH, _DHH), _f32),
            pltpu.VMEM((16,), _i32),
            pltpu.VMEM((_RPT + 1, _DHH), _f32),
            pltpu.SemaphoreType.DMA,
            pltpu.SemaphoreType.DMA,
        ],
    )(h2, lsrc, lldst, cnts)


def _dinv_from_parts(degp):
    deg = jnp.sum(degp, axis=0) + 1.0
    return lax.rsqrt(deg)


def _prep_kernel(x_ref, w1_ref, wl_ref, bl_ref, degp_ref, h1p_ref, hlin_ref):
    dinv = _dinv_from_parts(degp_ref[...])[:, None]
    xb = x_ref[...]
    h1 = jnp.dot(xb, w1_ref[...], preferred_element_type=_f32,
                 precision=lax.Precision.HIGHEST)
    h1p_ref[...] = h1 * dinv
    hlin_ref[...] = jnp.dot(xb, wl_ref[...], preferred_element_type=_f32,
                            precision=lax.Precision.HIGHEST) + bl_ref[...]


def _prep_call(x_p, W1, Wl, bl, degp):
    return pl.pallas_call(
        _prep_kernel,
        grid=(_GRID,),
        in_specs=[
            pl.BlockSpec((_RB, _DIN), lambda i: (i, 0)),
            pl.BlockSpec((_DIN, _DH), lambda i: (0, 0)),
            pl.BlockSpec((_DIN, _DH), lambda i: (0, 0)),
            pl.BlockSpec((1, _DH), lambda i: (0, 0)),
            pl.BlockSpec((_NT, _RB), lambda i: (0, i)),
        ],
        out_specs=[
            pl.BlockSpec((_RB, _DH), lambda i: (i, 0)),
            pl.BlockSpec((_RB, _DH), lambda i: (i, 0)),
        ],
        out_shape=[
            jax.ShapeDtypeStruct((_NPAD, _DH), _f32),
            jax.ShapeDtypeStruct((_NPAD, _DH), _f32),
        ],
    )(x_p, W1, Wl, bl, degp)


def _gate_kernel(Sa_ref, Sb_ref, hp_ref, res_ref, degp_ref, bc_ref, wg_ref,
                 bg_ref, hpre_ref, sum_ref, sq_ref, acc_s, acc_q):
    i = pl.program_id(0)
    dinv = _dinv_from_parts(degp_ref[...])[:, None]
    S = jnp.concatenate([Sa_ref[...], Sb_ref[...]], axis=1)
    z = jnp.tanh((S + hp_ref[...]) * dinv + bc_ref[...])
    g = jax.nn.sigmoid(jnp.dot(z, wg_ref[...], preferred_element_type=_f32,
                               precision=lax.Precision.HIGHEST) + bg_ref[...])
    h = jax.nn.relu((1.0 - g) * res_ref[...] + g * z)
    ridx = i * _RB + lax.broadcasted_iota(_i32, (_RB, 1), 0)
    h = jnp.where(ridx < _N, h, 0.0)
    hpre_ref[...] = h

    @pl.when(i == 0)
    def _():
        acc_s[...] = jnp.zeros_like(acc_s)
        acc_q[...] = jnp.zeros_like(acc_q)

    acc_s[...] += jnp.sum(h, axis=0, keepdims=True)
    acc_q[...] += jnp.sum(h * h, axis=0, keepdims=True)
    sum_ref[...] = acc_s[...]
    sq_ref[...] = acc_q[...]


def _gate_call(Sa, Sb, hp, res, degp, bc, Wg, bg):
    return pl.pallas_call(
        _gate_kernel,
        grid=(_GRID,),
        in_specs=[
            pl.BlockSpec((_RB, _DHH), lambda i: (i, 0)),
            pl.BlockSpec((_RB, _DHH), lambda i: (i, 0)),
            pl.BlockSpec((_RB, _DH), lambda i: (i, 0)),
            pl.BlockSpec((_RB, _DH), lambda i: (i, 0)),
            pl.BlockSpec((_NT, _RB), lambda i: (0, i)),
            pl.BlockSpec((1, _DH), lambda i: (0, 0)),
            pl.BlockSpec((_DH, _DH), lambda i: (0, 0)),
            pl.BlockSpec((1, _DH), lambda i: (0, 0)),
        ],
        out_specs=[
            pl.BlockSpec((_RB, _DH), lambda i: (i, 0)),
            pl.BlockSpec((1, _DH), lambda i: (0, 0)),
            pl.BlockSpec((1, _DH), lambda i: (0, 0)),
        ],
        out_shape=[
            jax.ShapeDtypeStruct((_NPAD, _DH), _f32),
            jax.ShapeDtypeStruct((1, _DH), _f32),
            jax.ShapeDtypeStruct((1, _DH), _f32),
        ],
        scratch_shapes=[
            pltpu.VMEM((1, _DH), _f32),
            pltpu.VMEM((1, _DH), _f32),
        ],
    )(Sa, Sb, hp, res, degp, bc, Wg, bg)


def _bnconv_kernel(hpre_ref, sc_ref, sh_ref, degp_ref, w2_ref,
                   hbn_ref, h2p_ref):
    dinv = _dinv_from_parts(degp_ref[...])[:, None]
    hbn = hpre_ref[...] * sc_ref[...] + sh_ref[...]
    i = pl.program_id(0)
    ridx = i * _RB + lax.broadcasted_iota(_i32, (_RB, 1), 0)
    hbn = jnp.where(ridx < _N, hbn, 0.0)
    hbn_ref[...] = hbn
    h2p_ref[...] = dinv * jnp.dot(hbn, w2_ref[...], preferred_element_type=_f32,
                                  precision=lax.Precision.HIGHEST)


def _bnconv_call(hpre, scale, shift, degp, W2):
    return pl.pallas_call(
        _bnconv_kernel,
        grid=(_GRID,),
        in_specs=[
            pl.BlockSpec((_RB, _DH), lambda i: (i, 0)),
            pl.BlockSpec((1, _DH), lambda i: (0, 0)),
            pl.BlockSpec((1, _DH), lambda i: (0, 0)),
            pl.BlockSpec((_NT, _RB), lambda i: (0, i)),
            pl.BlockSpec((_DH, _DH), lambda i: (0, 0)),
        ],
        out_specs=[
            pl.BlockSpec((_RB, _DH), lambda i: (i, 0)),
            pl.BlockSpec((_RB, _DH), lambda i: (i, 0)),
        ],
        out_shape=[
            jax.ShapeDtypeStruct((_NPAD, _DH), _f32),
            jax.ShapeDtypeStruct((_NPAD, _DH), _f32),
        ],
    )(hpre, scale, shift, degp, W2)


def _bnapply_kernel(hpre_ref, sc_ref, sh_ref, out_ref):
    out_ref[...] = hpre_ref[...] * sc_ref[...] + sh_ref[...]


def _bnapply_call(hpre, scale, shift):
    return pl.pallas_call(
        _bnapply_kernel,
        grid=(_GRID,),
        in_specs=[
            pl.BlockSpec((_RB, _DH), lambda i: (i, 0)),
            pl.BlockSpec((1, _DH), lambda i: (0, 0)),
            pl.BlockSpec((1, _DH), lambda i: (0, 0)),
        ],
        out_specs=pl.BlockSpec((_RB, _DH), lambda i: (i, 0)),
        out_shape=jax.ShapeDtypeStruct((_NPAD, _DH), _f32),
    )(hpre, scale, shift)


def kernel(x, edge_index, W_conv1, b_conv1, W_lin, b_lin, W_gate1, b_gate1,
           bn1_w, bn1_b, W_conv2, b_conv2, W_gate2, b_gate2, bn2_w, bn2_b):
    src = edge_index[0].astype(_i32)
    dst = edge_index[1].astype(_i32)
    pad = jnp.full((_EPAD - _E,), _N, _i32)
    src_p = jnp.concatenate([src, pad])
    dst_p = jnp.concatenate([dst, pad])
    x_p = jnp.pad(x, ((0, _NPAD - _N), (0, 0)))

    lsrc, lldst, cnts, degp = _index_call(src_p, dst_p)
    h1p, hlin = _prep_call(x_p, W_conv1, W_lin, b_lin.reshape(1, -1), degp)
    S1a, S1b = _scatter_call(h1p, lsrc, lldst, cnts)
    hpre, s1, q1 = _gate_call(S1a, S1b, h1p, hlin, degp,
                              b_conv1.reshape(1, -1), W_gate1,
                              b_gate1.reshape(1, -1))
    mu = s1 / _N
    var = q1 / _N - mu * mu
    scale1 = bn1_w.reshape(1, -1) * lax.rsqrt(var + 1e-5)
    shift1 = bn1_b.reshape(1, -1) - mu * scale1
    hbn, h2p = _bnconv_call(hpre, scale1, shift1, degp, W_conv2)
    S2a, S2b = _scatter_call(h2p, lsrc, lldst, cnts)
    hpre2, s2, q2 = _gate_call(S2a, S2b, h2p, hbn, degp,
                               b_conv2.reshape(1, -1), W_gate2,
                               b_gate2.reshape(1, -1))
    mu2 = s2 / _N
    var2 = q2 / _N - mu2 * mu2
    scale2 = bn2_w.reshape(1, -1) * lax.rsqrt(var2 + 1e-5)
    shift2 = bn2_b.reshape(1, -1) - mu2 * scale2
    out = _bnapply_call(hpre2, scale2, shift2)
    return out[:_N]


# speedup vs baseline: 4.1004x; 1.1809x over previous
import jax
import jax.numpy as jnp
from jax import lax
from jax.experimental import pallas as pl
from jax.experimental.pallas import tpu as pltpu
from jax.experimental.pallas import tpu_sc as plsc

_N = 10000
_E = 160000
_DIN = 256
_DH = 512
_DHH = _DH // 2

_NC, _NS = 2, 16
_NT = _NC * _NS
_NPAD = 10240
_EPAD = 163840
_SUB = 2560
_NSUB = _EPAD // _SUB
_RPT = _NPAD // _NT
_BATCH = 32
_LCAP = _EPAD + _NSUB * 8 + 64
_RB = 256
_GRID = _NPAD // _RB

_f32 = jnp.float32
_i32 = jnp.int32


def _mesh():
    return plsc.VectorSubcoreMesh(core_axis_name="c", subcore_axis_name="s")


def _index_body(src_hbm, dst_hbm, lsrc_hbm, lldst_hbm, cnt_hbm, parts_hbm,
                esrc_v, edst_v, ssrc_v, sdst_v, hist_v, cnt_v):
    c = lax.axis_index("c")
    s = lax.axis_index("s")
    t = c * _NS + s
    lo = t * _RPT

    def zh(i, carry):
        hist_v[pl.ds(i * 16, 16)] = jnp.zeros((16,), _f32)
        return carry

    lax.fori_loop(0, _NPAD // 16, zh, 0)

    onesf = jnp.ones((16,), _f32)
    pad_d = jnp.full((16,), _RPT, _i32)
    pad_s = jnp.zeros((16,), _i32)
    degr0 = t * (_NSUB // _NT)

    def round_body(r, off):
        off = pl.multiple_of(off, 8)
        rb = pl.multiple_of(r * _SUB, 8)
        pltpu.sync_copy(src_hbm.at[pl.ds(rb, _SUB)], esrc_v)
        pltpu.sync_copy(dst_hbm.at[pl.ds(rb, _SUB)], edst_v)

        @pl.when((r >= degr0) & (r < degr0 + _NSUB // _NT))
        def _():
            def dbody(i, carry):
                d = edst_v[pl.ds(i * 16, 16)]
                plsc.addupdate_scatter(hist_v, [d], onesf)
                return carry

            lax.fori_loop(0, _SUB // 16, dbody, 0)

        def fbody(i, loff):
            d = edst_v[pl.ds(i * 16, 16)]
            sv = esrc_v[pl.ds(i * 16, 16)]
            m = (d >= lo) & (d < lo + _RPT)
            plsc.store_compressed(ssrc_v.at[pl.ds(loff, 16)], sv, mask=m)
            plsc.store_compressed(sdst_v.at[pl.ds(loff, 16)], d - lo, mask=m)
            return loff + jnp.sum(jnp.where(m, 1, 0))

        loff = lax.fori_loop(0, _SUB // 16, fbody, 0)
        sdst_v[pl.ds(loff, 16)] = pad_d
        ssrc_v[pl.ds(loff, 16)] = pad_s
        loff8 = (loff + 7) // 8 * 8
        ob = pl.multiple_of(t * _LCAP + off, 8)
        pltpu.sync_copy(ssrc_v, lsrc_hbm.at[pl.ds(ob, _SUB + 16)])
        pltpu.sync_copy(sdst_v, lldst_hbm.at[pl.ds(ob, _SUB + 16)])
        return off + loff8

    cnt = lax.fori_loop(0, _NSUB, round_body, 0)

    for j in range(_BATCH // 16):
        sdst_v[pl.ds(j * 16, 16)] = pad_d
        ssrc_v[pl.ds(j * 16, 16)] = pad_s
    cb = pl.multiple_of(t * _LCAP + cnt, 8)
    pltpu.sync_copy(ssrc_v.at[pl.ds(0, _BATCH)],
                    lsrc_hbm.at[pl.ds(cb, _BATCH)])
    pltpu.sync_copy(sdst_v.at[pl.ds(0, _BATCH)],
                    lldst_hbm.at[pl.ds(cb, _BATCH)])

    cnt_v[pl.ds(0, 16)] = jnp.full((16,), cnt, _i32)
    pltpu.sync_copy(cnt_v.at[pl.ds(0, 8)],
                    cnt_hbm.at[pl.ds(pl.multiple_of(t * 8, 8), 8)])
    pltpu.sync_copy(hist_v, parts_hbm.at[t])


def _index_call(src_p, dst_p):
    return pl.kernel(
        _index_body,
        out_type=(
            jax.ShapeDtypeStruct((_NT * _LCAP,), _i32),
            jax.ShapeDtypeStruct((_NT * _LCAP,), _i32),
            jax.ShapeDtypeStruct((_NT * 8,), _i32),
            jax.ShapeDtypeStruct((_NT, _NPAD), _f32),
        ),
        mesh=_mesh(),
        compiler_params=pltpu.CompilerParams(needs_layout_passes=False),
        scratch_types=[
            pltpu.VMEM((_SUB,), _i32),
            pltpu.VMEM((_SUB,), _i32),
            pltpu.VMEM((_SUB + 16,), _i32),
            pltpu.VMEM((_SUB + 16,), _i32),
            pltpu.VMEM((_NPAD,), _f32),
            pltpu.VMEM((16,), _i32),
        ],
    )(src_p, dst_p)


def _scatter_body(h2_hbm, lsrc_hbm, lldst_hbm, cnt_hbm, outa_hbm, outb_hbm,
                  lsrc_v, lldst_v, idx0_v, idx1_v, rows0_v, rows1_v, cnt_v,
                  acc_v, sem0, sem1):
    c = lax.axis_index("c")
    s = lax.axis_index("s")
    t = c * _NS + s
    lo = t * _RPT
    idx_bufs = (idx0_v, idx1_v)
    rows_bufs = (rows0_v, rows1_v)
    sems = (sem0, sem1)

    pltpu.sync_copy(cnt_hbm.at[pl.ds(pl.multiple_of(t * 8, 8), 8)],
                    cnt_v.at[pl.ds(0, 8)])
    cnt = cnt_v[pl.ds(0, 16)][0]
    nblk = (cnt + _SUB - 1) // _SUB

    for half in range(2):
        out_h = outa_hbm if half == 0 else outb_hbm

        def za(i, carry):
            acc_v[i // (_DHH // 16), pl.ds((i % (_DHH // 16)) * 16, 16)] = (
                jnp.zeros((16,), _f32))
            return carry

        lax.fori_loop(0, (_RPT + 1) * (_DHH // 16), za, 0)

        def blk_body(blk, carry):
            lb = pl.multiple_of(t * _LCAP + blk * _SUB, 8)
            pltpu.sync_copy(lsrc_hbm.at[pl.ds(lb, _SUB)], lsrc_v)
            pltpu.sync_copy(lldst_hbm.at[pl.ds(lb, _SUB)],
                            lldst_v.at[pl.ds(0, _SUB)])
            size = jnp.minimum(_SUB, cnt - blk * _SUB)
            nb = (size + _BATCH - 1) // _BATCH

            def stage_start(b, par):
                ib, sm = idx_bufs[par], sems[par]
                for j in range(_BATCH // 16):
                    sv = lsrc_v[pl.ds(b * _BATCH + j * 16, 16)]
                    ib[pl.ds(j * 16, 16)] = sv * 2 + half
                pltpu.async_copy(h2_hbm.at[ib], rows_bufs[par], sm)

            def wait_process(b, par):
                rb2 = rows_bufs[par]
                pltpu.make_async_copy(h2_hbm.at[idx_bufs[par]], rb2,
                                      sems[par]).wait()

                @plsc.parallel_loop(0, _BATCH, 1, unroll=4)
                def _(e):
                    ldst = lldst_v[pl.ds(b * _BATCH + e, 16)][0]
                    for j in range(_DHH // 16):
                        plsc.addupdate(acc_v.at[ldst, pl.ds(j * 16, 16)],
                                       rb2[e, pl.ds(j * 16, 16)])

            stage_start(0, 0)
            ng = (nb + 1) // 2

            def gb(g, carry2):
                for par in range(2):
                    b = g * 2 + par

                    @pl.when(b < nb)
                    def _():
                        @pl.when(b + 1 < nb)
                        def _():
                            stage_start(b + 1, 1 - par)

                        wait_process(b, par)

                return carry2

            lax.fori_loop(0, ng, gb, 0)
            return carry

        lax.fori_loop(0, nblk, blk_body, 0)
        pltpu.sync_copy(acc_v.at[pl.ds(0, _RPT)],
                        out_h.at[pl.ds(pl.multiple_of(lo, 8), _RPT)])


def _scatter_call(h_p, lsrc, lldst, cnts):
    h2 = h_p.reshape(2 * _NPAD, _DHH)
    return pl.kernel(
        _scatter_body,
        out_type=(
            jax.ShapeDtypeStruct((_NPAD, _DHH), _f32),
            jax.ShapeDtypeStruct((_NPAD, _DHH), _f32),
        ),
        mesh=_mesh(),
        compiler_params=pltpu.CompilerParams(needs_layout_passes=False),
        scratch_types=[
            pltpu.VMEM((_SUB,), _i32),
            pltpu.VMEM((_SUB + 16,), _i32),
            pltpu.VMEM((_BATCH,), _i32),
            pltpu.VMEM((_BATCH,), _i32),
            pltpu.VMEM((_BATCH, _DHH), _f32),
            pltpu.VMEM((_BATCH, _DHH), _f32),
            pltpu.VMEM((16,), _i32),
            pltpu.VMEM((_RPT + 1, _DHH), _f32),
            pltpu.SemaphoreType.DMA,
            pltpu.SemaphoreType.DMA,
        ],
    )(h2, lsrc, lldst, cnts)


def _dinv_from_parts(degp):
    deg = jnp.sum(degp, axis=0) + 1.0
    return lax.rsqrt(deg)


def _prep_kernel(x_ref, w1_ref, wl_ref, bl_ref, degp_ref, h1p_ref, hlin_ref):
    dinv = _dinv_from_parts(degp_ref[...])[:, None]
    xb = x_ref[...]
    h1 = jnp.dot(xb, w1_ref[...], preferred_element_type=_f32,
                 precision=lax.Precision.HIGHEST)
    h1p_ref[...] = h1 * dinv
    hlin_ref[...] = jnp.dot(xb, wl_ref[...], preferred_element_type=_f32,
                            precision=lax.Precision.HIGHEST) + bl_ref[...]


def _prep_call(x_p, W1, Wl, bl, degp):
    return pl.pallas_call(
        _prep_kernel,
        grid=(_GRID,),
        in_specs=[
            pl.BlockSpec((_RB, _DIN), lambda i: (i, 0)),
            pl.BlockSpec((_DIN, _DH), lambda i: (0, 0)),
            pl.BlockSpec((_DIN, _DH), lambda i: (0, 0)),
            pl.BlockSpec((1, _DH), lambda i: (0, 0)),
            pl.BlockSpec((_NT, _RB), lambda i: (0, i)),
        ],
        out_specs=[
            pl.BlockSpec((_RB, _DH), lambda i: (i, 0)),
            pl.BlockSpec((_RB, _DH), lambda i: (i, 0)),
        ],
        out_shape=[
            jax.ShapeDtypeStruct((_NPAD, _DH), _f32),
            jax.ShapeDtypeStruct((_NPAD, _DH), _f32),
        ],
    )(x_p, W1, Wl, bl, degp)


def _gate_kernel(Sa_ref, Sb_ref, hp_ref, res_ref, degp_ref, bc_ref, wg_ref,
                 bg_ref, hpre_ref, sum_ref, sq_ref, acc_s, acc_q):
    i = pl.program_id(0)
    dinv = _dinv_from_parts(degp_ref[...])[:, None]
    S = jnp.concatenate([Sa_ref[...], Sb_ref[...]], axis=1)
    z = jnp.tanh((S + hp_ref[...]) * dinv + bc_ref[...])
    g = jax.nn.sigmoid(jnp.dot(z, wg_ref[...], preferred_element_type=_f32,
                               precision=lax.Precision.HIGHEST) + bg_ref[...])
    h = jax.nn.relu((1.0 - g) * res_ref[...] + g * z)
    ridx = i * _RB + lax.broadcasted_iota(_i32, (_RB, 1), 0)
    h = jnp.where(ridx < _N, h, 0.0)
    hpre_ref[...] = h

    @pl.when(i == 0)
    def _():
        acc_s[...] = jnp.zeros_like(acc_s)
        acc_q[...] = jnp.zeros_like(acc_q)

    acc_s[...] += jnp.sum(h, axis=0, keepdims=True)
    acc_q[...] += jnp.sum(h * h, axis=0, keepdims=True)
    sum_ref[...] = acc_s[...]
    sq_ref[...] = acc_q[...]


def _gate_call(Sa, Sb, hp, res, degp, bc, Wg, bg):
    return pl.pallas_call(
        _gate_kernel,
        grid=(_GRID,),
        in_specs=[
            pl.BlockSpec((_RB, _DHH), lambda i: (i, 0)),
            pl.BlockSpec((_RB, _DHH), lambda i: (i, 0)),
            pl.BlockSpec((_RB, _DH), lambda i: (i, 0)),
            pl.BlockSpec((_RB, _DH), lambda i: (i, 0)),
            pl.BlockSpec((_NT, _RB), lambda i: (0, i)),
            pl.BlockSpec((1, _DH), lambda i: (0, 0)),
            pl.BlockSpec((_DH, _DH), lambda i: (0, 0)),
            pl.BlockSpec((1, _DH), lambda i: (0, 0)),
        ],
        out_specs=[
            pl.BlockSpec((_RB, _DH), lambda i: (i, 0)),
            pl.BlockSpec((1, _DH), lambda i: (0, 0)),
            pl.BlockSpec((1, _DH), lambda i: (0, 0)),
        ],
        out_shape=[
            jax.ShapeDtypeStruct((_NPAD, _DH), _f32),
            jax.ShapeDtypeStruct((1, _DH), _f32),
            jax.ShapeDtypeStruct((1, _DH), _f32),
        ],
        scratch_shapes=[
            pltpu.VMEM((1, _DH), _f32),
            pltpu.VMEM((1, _DH), _f32),
        ],
    )(Sa, Sb, hp, res, degp, bc, Wg, bg)


def _bnconv_kernel(hpre_ref, sc_ref, sh_ref, degp_ref, w2_ref,
                   hbn_ref, h2p_ref):
    dinv = _dinv_from_parts(degp_ref[...])[:, None]
    hbn = hpre_ref[...] * sc_ref[...] + sh_ref[...]
    i = pl.program_id(0)
    ridx = i * _RB + lax.broadcasted_iota(_i32, (_RB, 1), 0)
    hbn = jnp.where(ridx < _N, hbn, 0.0)
    hbn_ref[...] = hbn
    h2p_ref[...] = dinv * jnp.dot(hbn, w2_ref[...], preferred_element_type=_f32,
                                  precision=lax.Precision.HIGHEST)


def _bnconv_call(hpre, scale, shift, degp, W2):
    return pl.pallas_call(
        _bnconv_kernel,
        grid=(_GRID,),
        in_specs=[
            pl.BlockSpec((_RB, _DH), lambda i: (i, 0)),
            pl.BlockSpec((1, _DH), lambda i: (0, 0)),
            pl.BlockSpec((1, _DH), lambda i: (0, 0)),
            pl.BlockSpec((_NT, _RB), lambda i: (0, i)),
            pl.BlockSpec((_DH, _DH), lambda i: (0, 0)),
        ],
        out_specs=[
            pl.BlockSpec((_RB, _DH), lambda i: (i, 0)),
            pl.BlockSpec((_RB, _DH), lambda i: (i, 0)),
        ],
        out_shape=[
            jax.ShapeDtypeStruct((_NPAD, _DH), _f32),
            jax.ShapeDtypeStruct((_NPAD, _DH), _f32),
        ],
    )(hpre, scale, shift, degp, W2)


def _bnapply_kernel(hpre_ref, sc_ref, sh_ref, out_ref):
    out_ref[...] = hpre_ref[...] * sc_ref[...] + sh_ref[...]


def _bnapply_call(hpre, scale, shift):
    return pl.pallas_call(
        _bnapply_kernel,
        grid=(_GRID,),
        in_specs=[
            pl.BlockSpec((_RB, _DH), lambda i: (i, 0)),
            pl.BlockSpec((1, _DH), lambda i: (0, 0)),
            pl.BlockSpec((1, _DH), lambda i: (0, 0)),
        ],
        out_specs=pl.BlockSpec((_RB, _DH), lambda i: (i, 0)),
        out_shape=jax.ShapeDtypeStruct((_NPAD, _DH), _f32),
    )(hpre, scale, shift)


def kernel(x, edge_index, W_conv1, b_conv1, W_lin, b_lin, W_gate1, b_gate1,
           bn1_w, bn1_b, W_conv2, b_conv2, W_gate2, b_gate2, bn2_w, bn2_b):
    src = edge_index[0].astype(_i32)
    dst = edge_index[1].astype(_i32)
    pad = jnp.full((_EPAD - _E,), _N, _i32)
    src_p = jnp.concatenate([src, pad])
    dst_p = jnp.concatenate([dst, pad])
    x_p = jnp.pad(x, ((0, _NPAD - _N), (0, 0)))

    lsrc, lldst, cnts, degp = _index_call(src_p, dst_p)
    h1p, hlin = _prep_call(x_p, W_conv1, W_lin, b_lin.reshape(1, -1), degp)
    S1a, S1b = _scatter_call(h1p, lsrc, lldst, cnts)
    hpre, s1, q1 = _gate_call(S1a, S1b, h1p, hlin, degp,
                              b_conv1.reshape(1, -1), W_gate1,
                              b_gate1.reshape(1, -1))
    mu = s1 / _N
    var = q1 / _N - mu * mu
    scale1 = bn1_w.reshape(1, -1) * lax.rsqrt(var + 1e-5)
    shift1 = bn1_b.reshape(1, -1) - mu * scale1
    hbn, h2p = _bnconv_call(hpre, scale1, shift1, degp, W_conv2)
    S2a, S2b = _scatter_call(h2p, lsrc, lldst, cnts)
    hpre2, s2, q2 = _gate_call(S2a, S2b, h2p, hbn, degp,
                               b_conv2.reshape(1, -1), W_gate2,
                               b_gate2.reshape(1, -1))
    mu2 = s2 / _N
    var2 = q2 / _N - mu2 * mu2
    scale2 = bn2_w.reshape(1, -1) * lax.rsqrt(var2 + 1e-5)
    shift2 = bn2_b.reshape(1, -1) - mu2 * scale2
    out = _bnapply_call(hpre2, scale2, shift2)
    return out[:_N]
